# baseline (speedup 1.0000x reference)
import functools

import jax
import jax.numpy as jnp
from jax.experimental import pallas as pl

_ALPHA = 0.1
import numpy as np

_IMIN = np.int32(-(2**31))
_IMAX = np.int32(2**31 - 1)


def _body(vm_ref, x_ref, am_ref, o_ref, *, bb, n, total_count):
    b = pl.program_id(0)
    nb = pl.num_programs(0)
    am = am_ref[...]
    vm3 = (vm_ref[...] > 0)[None]
    bits = jax.lax.bitcast_convert_type(am, jnp.int32)
    t = jnp.bitwise_and(bits, jnp.int32(~127))
    jidx = jax.lax.broadcasted_iota(jnp.int32, am.shape, 2)
    kmax = jnp.max(jnp.where(vm3, t - jidx, _IMIN), axis=2)
    kmin = jnp.min(jnp.where(vm3, t + jidx, _IMAX), axis=2)
    jmax = jnp.bitwise_and(-kmax, 127)
    jmin = jnp.bitwise_and(kmin, 127)

    xall = x_ref[...]
    xb = xall.astype(jnp.bfloat16)
    g = jax.lax.dot_general(xb, xb, (((2,), (2,)), ((0,), (0,))),
                            preferred_element_type=jnp.float32)
    r2 = jnp.sum(xall * xall, axis=2)
    u = r2[:, None, :] - 2.0 * g
    sp = jnp.sum(jnp.where(jidx == jmax[:, :, None], u, 0.0), axis=2)
    sn = jnp.sum(jnp.where(jidx == jmin[:, :, None], u, 0.0), axis=2)
    dp = jnp.sqrt(jnp.maximum(r2 + sp, 0.0))
    dn = jnp.sqrt(jnp.maximum(r2 + sn, 0.0))
    h = jnp.maximum(dp - dn + _ALPHA, 0.0)

    acc = jnp.where(b == 0, h, o_ref[...] + h)
    mean_bcast = jnp.full((bb, n), jnp.sum(acc) / total_count, jnp.float32)
    o_ref[...] = jnp.where(b == nb - 1, mean_bcast, acc)


@jax.jit
def kernel(x, annotator_matrix, num_dist_types, num_levels):
    b, n, d = x.shape
    m = n - 1
    i = jnp.arange(n)[:, None]
    j = jnp.arange(n)[None, :]
    same_block = ((i - 1) // num_levels) == (((j - 1) * num_dist_types) // m)
    valid = jnp.where(i == 0, j > 0, jnp.where(j == 0, True, ~same_block))
    vmask = valid.astype(jnp.float32)

    bb = 16
    grid = b // bb
    out = pl.pallas_call(
        functools.partial(_body, bb=bb, n=n, total_count=b * n),
        grid=(grid,),
        in_specs=[
            pl.BlockSpec((n, n), lambda g: (0, 0)),
            pl.BlockSpec((bb, n, d), lambda g: (g, 0, 0)),
            pl.BlockSpec((bb, n, n), lambda g: (g, 0, 0)),
        ],
        out_specs=pl.BlockSpec((bb, n), lambda g: (0, 0)),
        out_shape=jax.ShapeDtypeStruct((bb, n), jnp.float32),
    )(vmask, x, annotator_matrix)
    return out[0, 0]


# speedup vs baseline: 7.2120x; 7.2120x over previous
import functools

import jax
import jax.numpy as jnp
from jax.experimental import pallas as pl

_ALPHA = 0.1
import numpy as np

_IMIN = np.int32(-(2**31))
_IMAX = np.int32(2**31 - 1)


def _body(vm_ref, x_ref, am_ref, o_ref, *, bb, n, total_count):
    b = pl.program_id(0)
    nb = pl.num_programs(0)
    am = am_ref[...]
    vm3 = (vm_ref[...] > 0)[None]
    bits = jax.lax.bitcast_convert_type(am, jnp.int32)
    t = jnp.bitwise_and(bits, jnp.int32(~127))
    jidx = jax.lax.broadcasted_iota(jnp.int32, am.shape, 2)
    kmax = jnp.max(jnp.where(vm3, t - jidx, _IMIN), axis=2)
    kmin = jnp.min(jnp.where(vm3, t + jidx, _IMAX), axis=2)
    jmax = jnp.bitwise_and(-kmax, 127)
    jmin = jnp.bitwise_and(kmin, 127)

    xall = x_ref[...]
    xb = xall.astype(jnp.bfloat16)
    r2 = jnp.sum(xall * xall, axis=2, keepdims=True)
    ones = jnp.ones((bb, n, 1), jnp.bfloat16)
    xa = jnp.concatenate([xb, ones], axis=2)
    ya = jnp.concatenate([-2.0 * xb, r2.astype(jnp.bfloat16)], axis=2)
    u = jax.lax.dot_general(xa, ya, (((2,), (2,)), ((0,), (0,))),
                            preferred_element_type=jnp.float32)
    sp = jnp.sum(jnp.where(jidx == jmax[:, :, None], u, 0.0), axis=2)
    sn = jnp.sum(jnp.where(jidx == jmin[:, :, None], u, 0.0), axis=2)
    r2f = r2.reshape(bb, n)
    dp = jnp.sqrt(jnp.maximum(r2f + sp, 0.0))
    dn = jnp.sqrt(jnp.maximum(r2f + sn, 0.0))
    h = jnp.maximum(dp - dn + _ALPHA, 0.0)

    acc = jnp.where(b == 0, h, o_ref[...] + h)
    mean_bcast = jnp.full((bb, n), jnp.sum(acc) / total_count, jnp.float32)
    o_ref[...] = jnp.where(b == nb - 1, mean_bcast, acc)


@jax.jit
def kernel(x, annotator_matrix, num_dist_types, num_levels):
    b, n, d = x.shape
    m = n - 1
    i = jnp.arange(n)[:, None]
    j = jnp.arange(n)[None, :]
    same_block = ((i - 1) // num_levels) == (((j - 1) * num_dist_types) // m)
    valid = jnp.where(i == 0, j > 0, jnp.where(j == 0, True, ~same_block))
    vmask = valid.astype(jnp.float32)

    bb = 16
    grid = b // bb
    out = pl.pallas_call(
        functools.partial(_body, bb=bb, n=n, total_count=b * n),
        grid=(grid,),
        in_specs=[
            pl.BlockSpec((n, n), lambda g: (0, 0)),
            pl.BlockSpec((bb, n, d), lambda g: (g, 0, 0)),
            pl.BlockSpec((bb, n, n), lambda g: (g, 0, 0)),
        ],
        out_specs=pl.BlockSpec((bb, n), lambda g: (0, 0)),
        out_shape=jax.ShapeDtypeStruct((bb, n), jnp.float32),
    )(vmask, x, annotator_matrix)
    return out[0, 0]
